# baseline (speedup 1.0000x reference)
import functools

import jax
import jax.numpy as jnp
from jax import lax
from jax.experimental import pallas as pl
from jax.experimental.pallas import tpu as pltpu
from jax.experimental.pallas import tpu_sc as plsc

_NC, _NS = 2, 16
_NW = _NC * _NS
_CHUNK = 16 * 1024
_LANES = 16


def _sc_body(x_hbm, p_hbm, o_hbm, xb, pb, ob, sx, sp, so):
    total = x_hbm.shape[0]
    ptotal = p_hbm.shape[0]
    per_w = total // _NW
    nstep = per_w // _CHUNK
    wid = lax.axis_index("s") * _NC + lax.axis_index("c")
    base = wid * per_w
    pbase = lax.rem(base, ptotal)

    def in_copies(step, slot):
        r0 = base + step * _CHUNK
        q0 = pbase + step * _CHUNK
        return (
            pltpu.make_async_copy(x_hbm.at[pl.ds(r0, _CHUNK)], xb.at[slot], sx.at[slot]),
            pltpu.make_async_copy(p_hbm.at[pl.ds(q0, _CHUNK)], pb.at[slot], sp.at[slot]),
        )

    def out_copy(step, slot):
        r0 = base + step * _CHUNK
        return pltpu.make_async_copy(ob.at[slot], o_hbm.at[pl.ds(r0, _CHUNK)], so.at[slot])

    def compute(slot):
        def body(i, _):
            off = i * _LANES
            ob[slot, pl.ds(off, _LANES)] = (
                xb[slot, pl.ds(off, _LANES)] + pb[slot, pl.ds(off, _LANES)]
            )
            return _
        lax.fori_loop(0, _CHUNK // _LANES, body, None, unroll=8)

    for c in in_copies(0, 0):
        c.start()
    for s in range(nstep):
        slot = s % 2
        if s + 1 < nstep:
            for c in in_copies(s + 1, 1 - slot):
                c.start()
        for c in in_copies(s, slot):
            c.wait()
        if s >= 2:
            out_copy(s - 2, slot).wait()
        compute(slot)
        out_copy(s, slot).start()
    if nstep >= 2:
        out_copy(nstep - 2, 0 if nstep % 2 == 0 else 1).wait()
    out_copy(nstep - 1, 1 if nstep % 2 == 0 else 0).wait()


def _sc_add(x, pos_table):
    B, S, D = x.shape
    xf = x.reshape(B * S * D)
    pf = pos_table.reshape(S * D)
    run = pl.kernel(
        _sc_body,
        out_type=jax.ShapeDtypeStruct((B * S * D,), x.dtype),
        mesh=plsc.VectorSubcoreMesh(
            core_axis_name="c", subcore_axis_name="s",
            num_cores=_NC, num_subcores=_NS,
        ),
        scratch_types=[
            pltpu.VMEM((2, _CHUNK), jnp.float32),
            pltpu.VMEM((2, _CHUNK), jnp.float32),
            pltpu.VMEM((2, _CHUNK), jnp.float32),
            pltpu.SemaphoreType.DMA((2,)),
            pltpu.SemaphoreType.DMA((2,)),
            pltpu.SemaphoreType.DMA((2,)),
        ],
    )
    return run(xf, pf).reshape(B, S, D)


_BS = 2048


def _add_body(x_ref, p_ref, o_ref):
    o_ref[...] = x_ref[...] + p_ref[...]


def _tc_add(x, pos_table):
    B, S, D = x.shape
    bs = min(_BS, S)
    grid = (S // bs, B)
    return pl.pallas_call(
        _add_body,
        grid=grid,
        in_specs=[
            pl.BlockSpec((1, bs, D), lambda i, b: (b, i, 0)),
            pl.BlockSpec((bs, D), lambda i, b: (i, 0)),
        ],
        out_specs=pl.BlockSpec((1, bs, D), lambda i, b: (b, i, 0)),
        out_shape=jax.ShapeDtypeStruct(x.shape, x.dtype),
    )(x, pos_table)


def kernel(x, pos_table):
    return _sc_add(x, pos_table)


# speedup vs baseline: 1.5430x; 1.5430x over previous
import functools

import jax
import jax.numpy as jnp
from jax import lax
from jax.experimental import pallas as pl
from jax.experimental.pallas import tpu as pltpu
from jax.experimental.pallas import tpu_sc as plsc

_NC, _NS = 2, 16
_NW = _NC * _NS
_CHUNK = 16 * 1024
_LANES = 16


def _sc_body(x_hbm, p_hbm, o_hbm, xb, pb, ob, sx, sp, so):
    total = x_hbm.shape[0]
    ptotal = p_hbm.shape[0]
    per_w = total // _NW
    nstep = per_w // _CHUNK
    wid = lax.axis_index("s") * _NC + lax.axis_index("c")
    base = wid * per_w
    pbase = lax.rem(base, ptotal)

    def in_copies(step, slot):
        r0 = base + step * _CHUNK
        q0 = pbase + step * _CHUNK
        return (
            pltpu.make_async_copy(x_hbm.at[pl.ds(r0, _CHUNK)], xb.at[slot], sx.at[slot]),
            pltpu.make_async_copy(p_hbm.at[pl.ds(q0, _CHUNK)], pb.at[slot], sp.at[slot]),
        )

    def out_copy(step, slot):
        r0 = base + step * _CHUNK
        return pltpu.make_async_copy(ob.at[slot], o_hbm.at[pl.ds(r0, _CHUNK)], so.at[slot])

    def compute(slot):
        @plsc.parallel_loop(0, _CHUNK, step=_LANES, unroll=8)
        def _(off):
            ob[slot, pl.ds(off, _LANES)] = (
                xb[slot, pl.ds(off, _LANES)] + pb[slot, pl.ds(off, _LANES)]
            )

    for c in in_copies(0, 0):
        c.start()
    for s in range(nstep):
        slot = s % 2
        if s + 1 < nstep:
            for c in in_copies(s + 1, 1 - slot):
                c.start()
        for c in in_copies(s, slot):
            c.wait()
        if s >= 2:
            out_copy(s - 2, slot).wait()
        compute(slot)
        out_copy(s, slot).start()
    if nstep >= 2:
        out_copy(nstep - 2, 0 if nstep % 2 == 0 else 1).wait()
    out_copy(nstep - 1, 1 if nstep % 2 == 0 else 0).wait()


def _sc_add(x, pos_table):
    B, S, D = x.shape
    xf = x.reshape(B * S * D)
    pf = pos_table.reshape(S * D)
    run = pl.kernel(
        _sc_body,
        out_type=jax.ShapeDtypeStruct((B * S * D,), x.dtype),
        mesh=plsc.VectorSubcoreMesh(
            core_axis_name="c", subcore_axis_name="s",
            num_cores=_NC, num_subcores=_NS,
        ),
        scratch_types=[
            pltpu.VMEM((2, _CHUNK), jnp.float32),
            pltpu.VMEM((2, _CHUNK), jnp.float32),
            pltpu.VMEM((2, _CHUNK), jnp.float32),
            pltpu.SemaphoreType.DMA((2,)),
            pltpu.SemaphoreType.DMA((2,)),
            pltpu.SemaphoreType.DMA((2,)),
        ],
    )
    return run(xf, pf).reshape(B, S, D)


_BS = 2048


def _add_body(x_ref, p_ref, o_ref):
    o_ref[...] = x_ref[...] + p_ref[...]


def _tc_add(x, pos_table):
    B, S, D = x.shape
    bs = min(_BS, S)
    grid = (S // bs, B)
    return pl.pallas_call(
        _add_body,
        grid=grid,
        in_specs=[
            pl.BlockSpec((1, bs, D), lambda i, b: (b, i, 0)),
            pl.BlockSpec((bs, D), lambda i, b: (i, 0)),
        ],
        out_specs=pl.BlockSpec((1, bs, D), lambda i, b: (b, i, 0)),
        out_shape=jax.ShapeDtypeStruct(x.shape, x.dtype),
    )(x, pos_table)


def kernel(x, pos_table):
    return _sc_add(x, pos_table)


# speedup vs baseline: 4.4950x; 2.9131x over previous
import functools

import jax
import jax.numpy as jnp
from jax import lax
from jax.experimental import pallas as pl
from jax.experimental.pallas import tpu as pltpu
from jax.experimental.pallas import tpu_sc as plsc

_NC, _NS = 2, 16
_NW = _NC * _NS
_CHUNK = 16 * 1024
_LANES = 16


_ROWS = 16
_D = 1024


def _sc_body(x_hbm, p_hbm, o_hbm, xb, pb, ob, sx, sp, so):
    total = x_hbm.shape[0]
    ptotal = p_hbm.shape[0]
    per_w = total // _NW
    nstep = per_w // _ROWS
    wid = lax.axis_index("s") * _NC + lax.axis_index("c")
    base = wid * per_w
    pbase = lax.rem(base, ptotal)

    def in_copies(step, slot):
        r0 = base + step * _ROWS
        q0 = pbase + step * _ROWS
        return (
            pltpu.make_async_copy(x_hbm.at[pl.ds(r0, _ROWS)], xb.at[slot], sx.at[slot]),
            pltpu.make_async_copy(p_hbm.at[pl.ds(q0, _ROWS)], pb.at[slot], sp.at[slot]),
        )

    def out_copy(step, slot):
        r0 = base + step * _ROWS
        return pltpu.make_async_copy(ob.at[slot], o_hbm.at[pl.ds(r0, _ROWS)], so.at[slot])

    def compute(slot):
        @plsc.parallel_loop(0, _ROWS * _D, step=_LANES, unroll=8)
        def _(off):
            r = off // _D
            c = off % _D
            ob[slot, r, pl.ds(c, _LANES)] = (
                xb[slot, r, pl.ds(c, _LANES)] + pb[slot, r, pl.ds(c, _LANES)]
            )

    for c in in_copies(0, 0):
        c.start()
    for s in range(nstep):
        slot = s % 2
        if s + 1 < nstep:
            for c in in_copies(s + 1, 1 - slot):
                c.start()
        for c in in_copies(s, slot):
            c.wait()
        if s >= 2:
            out_copy(s - 2, slot).wait()
        compute(slot)
        out_copy(s, slot).start()
    if nstep >= 2:
        out_copy(nstep - 2, 0 if nstep % 2 == 0 else 1).wait()
    out_copy(nstep - 1, 1 if nstep % 2 == 0 else 0).wait()


def _sc_add(x, pos_table):
    B, S, D = x.shape
    xf = x.reshape(B * S, D)
    pf = pos_table
    run = pl.kernel(
        _sc_body,
        out_type=jax.ShapeDtypeStruct((B * S, D), x.dtype),
        mesh=plsc.VectorSubcoreMesh(
            core_axis_name="c", subcore_axis_name="s",
            num_cores=_NC, num_subcores=_NS,
        ),
        scratch_types=[
            pltpu.VMEM((2, _ROWS, _D), jnp.float32),
            pltpu.VMEM((2, _ROWS, _D), jnp.float32),
            pltpu.VMEM((2, _ROWS, _D), jnp.float32),
            pltpu.SemaphoreType.DMA((2,)),
            pltpu.SemaphoreType.DMA((2,)),
            pltpu.SemaphoreType.DMA((2,)),
        ],
    )
    return run(xf, pf).reshape(B, S, D)


_BS = 2048


def _add_body(x_ref, p_ref, o_ref):
    o_ref[...] = x_ref[...] + p_ref[...]


def _tc_add(x, pos_table):
    B, S, D = x.shape
    bs = min(_BS, S)
    grid = (S // bs, B)
    return pl.pallas_call(
        _add_body,
        grid=grid,
        in_specs=[
            pl.BlockSpec((1, bs, D), lambda i, b: (b, i, 0)),
            pl.BlockSpec((bs, D), lambda i, b: (i, 0)),
        ],
        out_specs=pl.BlockSpec((1, bs, D), lambda i, b: (b, i, 0)),
        out_shape=jax.ShapeDtypeStruct(x.shape, x.dtype),
    )(x, pos_table)


def kernel(x, pos_table):
    return _sc_add(x, pos_table)


# speedup vs baseline: 5.6899x; 1.2658x over previous
import functools

import jax
import jax.numpy as jnp
from jax import lax
from jax.experimental import pallas as pl
from jax.experimental.pallas import tpu as pltpu
from jax.experimental.pallas import tpu_sc as plsc

_NC, _NS = 2, 16
_NW = _NC * _NS
_CHUNK = 16 * 1024
_LANES = 16


_ROWS = 8
_D = 1024
_B = 4


def _sc_body(x_hbm, p_hbm, o_hbm, xb, pb, sx, sp, so):
    S = p_hbm.shape[0]
    spw = S // _NW
    nstep = spw // _ROWS
    wid = lax.axis_index("s") * _NC + lax.axis_index("c")
    s_base = wid * spw

    def in_copies(step, slot):
        s0 = s_base + step * _ROWS
        cps = [pltpu.make_async_copy(
            p_hbm.at[pl.ds(s0, _ROWS)], pb.at[slot], sp.at[slot])]
        for b in range(_B):
            cps.append(pltpu.make_async_copy(
                x_hbm.at[pl.ds(b * S + s0, _ROWS)], xb.at[slot, b], sx.at[slot]))
        return cps

    def out_copies(step, slot):
        s0 = s_base + step * _ROWS
        return [pltpu.make_async_copy(
            xb.at[slot, b], o_hbm.at[pl.ds(b * S + s0, _ROWS)], so.at[slot])
            for b in range(_B)]

    def compute(slot):
        @plsc.parallel_loop(0, _ROWS * _D, step=_LANES, unroll=4)
        def _(off):
            r = off // _D
            c = off % _D
            pv = pb[slot, r, pl.ds(c, _LANES)]
            for b in range(_B):
                xb[slot, b, r, pl.ds(c, _LANES)] = (
                    xb[slot, b, r, pl.ds(c, _LANES)] + pv
                )

    for c in in_copies(0, 0):
        c.start()
    for s in range(nstep):
        slot = s % 2
        if s + 1 < nstep:
            if s >= 1:
                for c in out_copies(s - 1, 1 - slot):
                    c.wait()
            for c in in_copies(s + 1, 1 - slot):
                c.start()
        for c in in_copies(s, slot):
            c.wait()
        compute(slot)
        for c in out_copies(s, slot):
            c.start()
    if nstep >= 2:
        for c in out_copies(nstep - 2, (nstep - 2) % 2):
            c.wait()
    for c in out_copies(nstep - 1, (nstep - 1) % 2):
        c.wait()


def _sc_add(x, pos_table):
    B, S, D = x.shape
    xf = x.reshape(B * S, D)
    pf = pos_table
    run = pl.kernel(
        _sc_body,
        out_type=jax.ShapeDtypeStruct((B * S, D), x.dtype),
        mesh=plsc.VectorSubcoreMesh(
            core_axis_name="c", subcore_axis_name="s",
            num_cores=_NC, num_subcores=_NS,
        ),
        scratch_types=[
            pltpu.VMEM((2, _B, _ROWS, _D), jnp.float32),
            pltpu.VMEM((2, _ROWS, _D), jnp.float32),
            pltpu.SemaphoreType.DMA((2,)),
            pltpu.SemaphoreType.DMA((2,)),
            pltpu.SemaphoreType.DMA((2,)),
        ],
    )
    return run(xf, pf).reshape(B, S, D)


_BS = 2048


def _add_body(x_ref, p_ref, o_ref):
    o_ref[...] = x_ref[...] + p_ref[...]


def _tc_add(x, pos_table):
    B, S, D = x.shape
    bs = min(_BS, S)
    grid = (S // bs, B)
    return pl.pallas_call(
        _add_body,
        grid=grid,
        in_specs=[
            pl.BlockSpec((1, bs, D), lambda i, b: (b, i, 0)),
            pl.BlockSpec((bs, D), lambda i, b: (i, 0)),
        ],
        out_specs=pl.BlockSpec((1, bs, D), lambda i, b: (b, i, 0)),
        out_shape=jax.ShapeDtypeStruct(x.shape, x.dtype),
    )(x, pos_table)


def kernel(x, pos_table):
    return _sc_add(x, pos_table)
